# baseline (speedup 1.0000x reference)
import functools

import jax
import jax.numpy as jnp
from jax import lax
from jax.experimental import pallas as pl
from jax.experimental.pallas import tpu as pltpu
from jax.experimental.pallas import tpu_sc as plsc

NNODE = 10000
NEDGE = 320000
NEG = 0.2

NP = 10240
BE = 128
NB = 162
EPT = BE * NB
EP = 16 * EPT
RPT = NP // 16


def _d1_body(x_ref, w_ref, a_ref, tbl_ref, asad_ref):
    h = jnp.dot(x_ref[...], w_ref[...], preferred_element_type=jnp.float32)
    asad_ref[...] = jnp.dot(h, a_ref[...], preferred_element_type=jnp.float32)
    for q in range(8):
        tbl_ref[q, :, :] = h[:, q * 32:(q + 1) * 32]


def _dense1(x_pad, w1, a1):
    bm = 512
    return pl.pallas_call(
        _d1_body,
        grid=(NP // bm,),
        in_specs=[
            pl.BlockSpec((bm, 128), lambda i: (i, 0)),
            pl.BlockSpec((128, 256), lambda i: (0, 0)),
            pl.BlockSpec((256, 16), lambda i: (0, 0)),
        ],
        out_specs=[
            pl.BlockSpec((8, bm, 32), lambda i: (0, i, 0)),
            pl.BlockSpec((bm, 16), lambda i: (i, 0)),
        ],
        out_shape=[
            jax.ShapeDtypeStruct((8, NP, 32), jnp.float32),
            jax.ShapeDtypeStruct((NP, 16), jnp.float32),
        ],
    )(x_pad, w1, a1)


def _d2_body(acc_ref, den_ref, b1_ref, w_ref, a_ref, tbl_ref, asad_ref):
    parts = []
    for q in range(8):
        feat = acc_ref[q, :, :]
        den = jnp.sum(den_ref[q], axis=0)[:, None]
        parts.append(feat / (den + 1e-16))
    u = jnp.concatenate(parts, axis=1) + b1_ref[...]
    u = jnp.where(u > 0, u, jnp.exp(jnp.minimum(u, 0.0)) - 1.0)
    h2 = jnp.dot(u, w_ref[...], preferred_element_type=jnp.float32)
    asad_ref[...] = jnp.dot(h2, a_ref[...], preferred_element_type=jnp.float32)
    tbl_ref[0, :, :] = h2[:, 0:32]
    tbl_ref[1, :, :] = h2[:, 32:64]


def _dense2(acc1, den1, b1, w2, a2):
    bm = 512
    return pl.pallas_call(
        _d2_body,
        grid=(NP // bm,),
        in_specs=[
            pl.BlockSpec((8, bm, 32), lambda i: (0, i, 0)),
            pl.BlockSpec((8, 16, bm), lambda i: (0, 0, i)),
            pl.BlockSpec((1, 256), lambda i: (0, 0)),
            pl.BlockSpec((256, 64), lambda i: (0, 0)),
            pl.BlockSpec((64, 2), lambda i: (0, 0)),
        ],
        out_specs=[
            pl.BlockSpec((2, bm, 32), lambda i: (0, i, 0)),
            pl.BlockSpec((bm, 2), lambda i: (i, 0)),
        ],
        out_shape=[
            jax.ShapeDtypeStruct((2, NP, 32), jnp.float32),
            jax.ShapeDtypeStruct((NP, 2), jnp.float32),
        ],
    )(acc1, den1, b1, w2, a2)


def _fin_body(acc_ref, den_ref, b2_ref, o_ref):
    d0 = jnp.sum(den_ref[0], axis=0)[:, None]
    d1 = jnp.sum(den_ref[1], axis=0)[:, None]
    left = acc_ref[0, :, :] / (d0 + 1e-16)
    right = acc_ref[1, :, :] / (d1 + 1e-16)
    o_ref[...] = jnp.concatenate([left, right], axis=1) + b2_ref[...]


def _finalize(acc2, den2, b2):
    bm = 512
    return pl.pallas_call(
        _fin_body,
        grid=(NP // bm,),
        in_specs=[
            pl.BlockSpec((2, bm, 32), lambda i: (0, i, 0)),
            pl.BlockSpec((2, 16, bm), lambda i: (0, 0, i)),
            pl.BlockSpec((1, 64), lambda i: (0, 0)),
        ],
        out_specs=pl.BlockSpec((bm, 64), lambda i: (i, 0)),
        out_shape=jax.ShapeDtypeStruct((NP, 64), jnp.float32),
    )(acc2, den2, b2)


def _make_edge_kernel(fh, hh, fr, nphase):
    nfv = fh // 16
    mesh = plsc.VectorSubcoreMesh(core_axis_name="c", subcore_axis_name="s")

    @functools.partial(
        pl.kernel,
        mesh=mesh,
        out_type=(jax.ShapeDtypeStruct((2 * nphase, NP, fr), jnp.float32),
                  jax.ShapeDtypeStruct((2 * nphase, 16, NP * hh),
                                       jnp.float32)),
        compiler_params=pltpu.CompilerParams(
            needs_layout_passes=False, use_tc_tiling_on_sc=False),
        scratch_types=[
            pltpu.VMEM((NB, BE), jnp.int32),
            pltpu.VMEM((NB, BE), jnp.int32),
            pltpu.VMEM((NP * hh,), jnp.float32),
            pltpu.VMEM((NP * hh,), jnp.float32),
            pltpu.VMEM((hh, 16), jnp.float32),
            pltpu.VMEM((BE, fr), jnp.float32),
            pltpu.VMEM((BE, fr), jnp.float32),
            pltpu.VMEM((BE, fr), jnp.float32),
            pltpu.VMEM((BE, fr), jnp.float32),
            pltpu.VMEM((BE, fr), jnp.float32),
            pltpu.VMEM((16 * BE,), jnp.float32),
            pltpu.VMEM((NP * hh,), jnp.float32),
            pltpu.VMEM((64, fr), jnp.float32),
            pltpu.VMEM_SHARED((NP, fr), jnp.float32),
            pltpu.SemaphoreType.DMA,
            pltpu.SemaphoreType.DMA,
            pltpu.SemaphoreType.DMA,
            pltpu.SemaphoreType.DMA,
            pltpu.SemaphoreType.DMA,
        ],
    )
    def edge_kernel(srcg_hbm, dst_hbm, tbl_hbm, as_hbm, ad_hbm, msp_hbm,
                    out_hbm, outden_hbm, srcc_v, dstc_v, as_v, ad_v, msp_v,
                    gb0, gb1, gb2, sb0, sb1, exb_v, den_v, zb_v, acc,
                    gsem0, gsem1, gsem2, ssem0, ssem1):
        c = lax.axis_index("c")
        s = lax.axis_index("s")
        gbufs, sbufs = (gb0, gb1, gb2), (sb0, sb1)
        gsems, ssems = (gsem0, gsem1, gsem2), (ssem0, ssem1)
        zv = jnp.zeros((16,), jnp.float32)
        for r in range(64):
            for k in range(fr // 16):
                zb_v[r, pl.ds(k * 16, 16)] = zv
        pltpu.sync_copy(dst_hbm.at[s], dstc_v)
        pltpu.sync_copy(srcg_hbm.at[s], srcc_v)
        iot = jnp.arange(16, dtype=jnp.int32)
        nrv = BE // 16

        def bias_body(bias):
            def body(i, carry):
                for k in range(nrv):
                    srcc_v[i, pl.ds(k * 16, 16)] = (
                        srcc_v[i, pl.ds(k * 16, 16)] + bias)
                return carry
            lax.fori_loop(0, NB, body, 0)

        bias_body(c * NP)

        for p in range(nphase):
            q = 2 * p + c
            if p:
                bias_body(2 * NP)
            for i in range(RPT // 64):
                pltpu.sync_copy(zb_v, acc.at[pl.ds(s * RPT + i * 64, 64)])

            def dz_body(i, carry):
                den_v[pl.ds(i * 16, 16)] = jnp.zeros((16,), jnp.float32)
                return carry

            lax.fori_loop(0, NP * hh // 16, dz_body, 0)
            pltpu.sync_copy(as_hbm.at[q], as_v)
            pltpu.sync_copy(ad_hbm.at[q], ad_v)
            pltpu.sync_copy(msp_hbm.at[q], msp_v)
            plsc.subcore_barrier()

            pltpu.async_copy(tbl_hbm.at[srcc_v.at[0]], gb0, gsem0)
            pltpu.async_copy(tbl_hbm.at[srcc_v.at[1]], gb1, gsem1)
            pltpu.async_copy(tbl_hbm.at[srcc_v.at[2]], gb2, gsem2)

            def outer(g, carry):
                for b6 in range(6):
                    i = g * 6 + b6
                    b = b6 % 2
                    gbuf, gsem = gbufs[b6 % 3], gsems[b6 % 3]
                    for grp in range(BE // 16):
                        sg = srcc_v[i, pl.ds(grp * 16, 16)] - q * NP
                        dg = dstc_v[i, pl.ds(grp * 16, 16)]
                        for h in range(hh):
                            asg = plsc.load_gather(as_v, [sg * hh + h])
                            adg = plsc.load_gather(ad_v, [dg * hh + h])
                            al = asg + adg
                            al = jnp.maximum(al, NEG * al)
                            exv = jnp.exp(al - msp_v[h, :])
                            exb_v[pl.ds(h * BE + grp * 16, 16)] = exv
                            plsc.addupdate_scatter(
                                den_v, [dg * hh + h], exv)

                    sbuf, ssem = sbufs[b], ssems[b]
                    pltpu.make_async_copy(
                        tbl_hbm.at[srcc_v.at[i]], gbuf, gsem).wait()

                    @pl.when(i >= 2)
                    def _():
                        pltpu.make_async_copy(
                            sbuf, acc.at[dstc_v.at[i]], ssem).wait()

                    def e_body(j, ecarry):
                        for u in range(4):
                            e = j * 4 + u
                            svs = [
                                plsc.load_gather(
                                    exb_v,
                                    [jnp.full((16,), h * BE, jnp.int32) + e])
                                for h in range(hh)
                            ]
                            for k in range(nfv):
                                hsel = k // 2 if hh > 1 else 0
                                sbuf[e, pl.ds(k * 16, 16)] = (
                                    gbuf[e, pl.ds(k * 16, 16)] * svs[hsel])
                        return ecarry

                    lax.fori_loop(0, BE // 4, e_body, 0)

                    @pl.when(i + 3 < NB)
                    def _():
                        pltpu.async_copy(
                            tbl_hbm.at[srcc_v.at[i + 3]], gbuf, gsem)
                    pltpu.async_copy(
                        sbuf, acc.at[dstc_v.at[i]], ssem, add=True)
                return carry

            lax.fori_loop(0, NB // 6, outer, 0)
            for b in range(2):
                pltpu.make_async_copy(
                    sbufs[b], acc.at[dstc_v.at[b]], ssems[b]).wait()
            pltpu.sync_copy(den_v, outden_hbm.at[q, s])
            plsc.subcore_barrier()
            pltpu.sync_copy(acc.at[pl.ds(s * RPT, RPT)],
                            out_hbm.at[q, pl.ds(s * RPT, RPT)])

    return edge_kernel


_edge_l1 = _make_edge_kernel(32, 1, 32, 4)
_edge_l2 = _make_edge_kernel(32, 1, 32, 1)


def _lrelu_scalar(m):
    return jnp.where(m > 0, m, NEG * m)


def kernel(x, edge_index, W1, a_src1, a_dst1, b1, W2, a_src2, a_dst2, b2):
    f32 = jnp.float32
    loop = jnp.arange(NNODE, dtype=jnp.int32)
    padi = jnp.full((EP - NNODE - NEDGE,), NNODE, jnp.int32)
    src = jnp.concatenate([edge_index[0].astype(jnp.int32), loop, padi])
    dst = jnp.concatenate([edge_index[1].astype(jnp.int32), loop, padi])
    srcr = src.reshape(16, NB, BE)
    dstg = dst.reshape(16, NB, BE)

    heads = jnp.arange(256, dtype=jnp.int32) // 32
    oh = (heads[:, None] == jnp.arange(8)[None, :]).astype(f32)
    a1 = jnp.concatenate(
        [oh * a_src1.reshape(-1, 1), oh * a_dst1.reshape(-1, 1)], axis=1)
    x_pad = jnp.pad(x, ((0, NP - NNODE), (0, 0)))
    tbl1, asad1 = _dense1(x_pad, W1, a1)
    as1 = asad1[:, 0:8]
    ad1 = asad1[:, 8:16]
    lm1 = _lrelu_scalar(jnp.max(as1, axis=0) + jnp.max(ad1, axis=0))
    msp1 = jnp.broadcast_to(lm1.reshape(8, 1, 1), (8, 1, 16))
    as_t1 = as1.transpose(1, 0)
    ad_t1 = ad1.transpose(1, 0)
    tblf1 = tbl1.reshape(8 * NP, 32)

    acc1, den1 = _edge_l1(srcr, dstg, tblf1, as_t1, ad_t1, msp1)

    a2 = jnp.concatenate(
        [a_src2.reshape(-1, 1), a_dst2.reshape(-1, 1)], axis=1)
    tbl2, asad2 = _dense2(acc1, den1, b1.reshape(1, 256), W2, a2)
    as2 = asad2[:, 0]
    ad2 = asad2[:, 1]
    lm2 = _lrelu_scalar(jnp.max(as2) + jnp.max(ad2)).reshape(1)
    msp2 = jnp.broadcast_to(lm2.reshape(1, 1, 1), (2, 1, 16))
    as_t2 = jnp.broadcast_to(as2.reshape(1, NP), (2, NP))
    ad_t2 = jnp.broadcast_to(ad2.reshape(1, NP), (2, NP))

    acc2, den2 = _edge_l2(srcr, dstg, tbl2.reshape(2 * NP, 32),
                          as_t2, ad_t2, msp2)

    return _finalize(acc2, den2, b2.reshape(1, 64))[:NNODE]


# speedup vs baseline: 1.0090x; 1.0090x over previous
import functools

import jax
import jax.numpy as jnp
from jax import lax
from jax.experimental import pallas as pl
from jax.experimental.pallas import tpu as pltpu
from jax.experimental.pallas import tpu_sc as plsc

NNODE = 10000
NEDGE = 320000
NEG = 0.2

NP = 10240
BE = 128
NB = 162
EPT = BE * NB
EP = 16 * EPT
RPT = NP // 16


def _d1_body(x_ref, w_ref, a_ref, tbl_ref, asad_ref):
    h = jnp.dot(x_ref[...], w_ref[...], preferred_element_type=jnp.float32)
    asad_ref[...] = jnp.dot(h, a_ref[...], preferred_element_type=jnp.float32)
    for q in range(8):
        tbl_ref[q, :, :] = h[:, q * 32:(q + 1) * 32]


def _dense1(x_pad, w1, a1):
    bm = 512
    return pl.pallas_call(
        _d1_body,
        grid=(NP // bm,),
        in_specs=[
            pl.BlockSpec((bm, 128), lambda i: (i, 0)),
            pl.BlockSpec((128, 256), lambda i: (0, 0)),
            pl.BlockSpec((256, 16), lambda i: (0, 0)),
        ],
        out_specs=[
            pl.BlockSpec((8, bm, 32), lambda i: (0, i, 0)),
            pl.BlockSpec((bm, 16), lambda i: (i, 0)),
        ],
        out_shape=[
            jax.ShapeDtypeStruct((8, NP, 32), jnp.float32),
            jax.ShapeDtypeStruct((NP, 16), jnp.float32),
        ],
    )(x_pad, w1, a1)


def _d2_body(acc_ref, den_ref, b1_ref, w_ref, a_ref, tbl_ref, asad_ref):
    parts = []
    for q in range(8):
        feat = acc_ref[q, :, :]
        den = jnp.sum(den_ref[q], axis=0)[:, None]
        parts.append(feat / (den + 1e-16))
    u = jnp.concatenate(parts, axis=1) + b1_ref[...]
    u = jnp.where(u > 0, u, jnp.exp(jnp.minimum(u, 0.0)) - 1.0)
    h2 = jnp.dot(u, w_ref[...], preferred_element_type=jnp.float32)
    asad_ref[...] = jnp.dot(h2, a_ref[...], preferred_element_type=jnp.float32)
    tbl_ref[0, :, :] = h2[:, 0:32]
    tbl_ref[1, :, :] = h2[:, 32:64]


def _dense2(acc1, den1, b1, w2, a2):
    bm = 512
    return pl.pallas_call(
        _d2_body,
        grid=(NP // bm,),
        in_specs=[
            pl.BlockSpec((8, bm, 32), lambda i: (0, i, 0)),
            pl.BlockSpec((8, 16, bm), lambda i: (0, 0, i)),
            pl.BlockSpec((1, 256), lambda i: (0, 0)),
            pl.BlockSpec((256, 64), lambda i: (0, 0)),
            pl.BlockSpec((64, 2), lambda i: (0, 0)),
        ],
        out_specs=[
            pl.BlockSpec((2, bm, 32), lambda i: (0, i, 0)),
            pl.BlockSpec((bm, 2), lambda i: (i, 0)),
        ],
        out_shape=[
            jax.ShapeDtypeStruct((2, NP, 32), jnp.float32),
            jax.ShapeDtypeStruct((NP, 2), jnp.float32),
        ],
    )(acc1, den1, b1, w2, a2)


def _fin_body(acc_ref, den_ref, b2_ref, o_ref):
    d0 = jnp.sum(den_ref[0], axis=0)[:, None]
    d1 = jnp.sum(den_ref[1], axis=0)[:, None]
    left = acc_ref[0, :, :] / (d0 + 1e-16)
    right = acc_ref[1, :, :] / (d1 + 1e-16)
    o_ref[...] = jnp.concatenate([left, right], axis=1) + b2_ref[...]


def _finalize(acc2, den2, b2):
    bm = 512
    return pl.pallas_call(
        _fin_body,
        grid=(NP // bm,),
        in_specs=[
            pl.BlockSpec((2, bm, 32), lambda i: (0, i, 0)),
            pl.BlockSpec((2, 16, bm), lambda i: (0, 0, i)),
            pl.BlockSpec((1, 64), lambda i: (0, 0)),
        ],
        out_specs=pl.BlockSpec((bm, 64), lambda i: (i, 0)),
        out_shape=jax.ShapeDtypeStruct((NP, 64), jnp.float32),
    )(acc2, den2, b2)


def _make_edge_kernel(fh, hh, fr, nphase):
    nfv = fh // 16
    mesh = plsc.VectorSubcoreMesh(core_axis_name="c", subcore_axis_name="s")

    @functools.partial(
        pl.kernel,
        mesh=mesh,
        out_type=(jax.ShapeDtypeStruct((2 * nphase, NP, fr), jnp.float32),
                  jax.ShapeDtypeStruct((2 * nphase, 16, NP * hh),
                                       jnp.float32)),
        compiler_params=pltpu.CompilerParams(
            needs_layout_passes=False, use_tc_tiling_on_sc=False),
        scratch_types=[
            pltpu.VMEM((NB, BE), jnp.int32),
            pltpu.VMEM((NB, BE), jnp.int32),
            pltpu.VMEM((NP * hh,), jnp.float32),
            pltpu.VMEM((NP * hh,), jnp.float32),
            pltpu.VMEM((hh, 16), jnp.float32),
            pltpu.VMEM((BE, fr), jnp.float32),
            pltpu.VMEM((BE, fr), jnp.float32),
            pltpu.VMEM((BE, fr), jnp.float32),
            pltpu.VMEM((BE, fr), jnp.float32),
            pltpu.VMEM((16 * BE,), jnp.float32),
            pltpu.VMEM((NP * hh,), jnp.float32),
            pltpu.VMEM((64, fr), jnp.float32),
            pltpu.VMEM_SHARED((NP, fr), jnp.float32),
            pltpu.SemaphoreType.DMA,
            pltpu.SemaphoreType.DMA,
            pltpu.SemaphoreType.DMA,
            pltpu.SemaphoreType.DMA,
        ],
    )
    def edge_kernel(srcg_hbm, dst_hbm, tbl_hbm, as_hbm, ad_hbm, msp_hbm,
                    out_hbm, outden_hbm, srcc_v, dstc_v, as_v, ad_v, msp_v,
                    gb0, gb1, sb0, sb1, exb_v, den_v, zb_v, acc,
                    gsem0, gsem1, ssem0, ssem1):
        c = lax.axis_index("c")
        s = lax.axis_index("s")
        gbufs, sbufs = (gb0, gb1), (sb0, sb1)
        gsems, ssems = (gsem0, gsem1), (ssem0, ssem1)
        zv = jnp.zeros((16,), jnp.float32)
        for r in range(64):
            for k in range(fr // 16):
                zb_v[r, pl.ds(k * 16, 16)] = zv
        pltpu.sync_copy(dst_hbm.at[s], dstc_v)
        pltpu.sync_copy(srcg_hbm.at[s], srcc_v)
        iot = jnp.arange(16, dtype=jnp.int32)
        nrv = BE // 16

        def bias_body(bias):
            def body(i, carry):
                for k in range(nrv):
                    srcc_v[i, pl.ds(k * 16, 16)] = (
                        srcc_v[i, pl.ds(k * 16, 16)] + bias)
                return carry
            lax.fori_loop(0, NB, body, 0)

        bias_body(c * NP)

        for p in range(nphase):
            q = 2 * p + c
            if p:
                bias_body(2 * NP)
            for i in range(RPT // 64):
                pltpu.sync_copy(zb_v, acc.at[pl.ds(s * RPT + i * 64, 64)])

            def dz_body(i, carry):
                den_v[pl.ds(i * 16, 16)] = jnp.zeros((16,), jnp.float32)
                return carry

            lax.fori_loop(0, NP * hh // 16, dz_body, 0)
            pltpu.sync_copy(as_hbm.at[q], as_v)
            pltpu.sync_copy(ad_hbm.at[q], ad_v)
            pltpu.sync_copy(msp_hbm.at[q], msp_v)
            plsc.subcore_barrier()

            pltpu.async_copy(tbl_hbm.at[srcc_v.at[0]], gb0, gsem0)
            pltpu.async_copy(tbl_hbm.at[srcc_v.at[1]], gb1, gsem1)

            def outer(g, carry):
                for b in range(2):
                    i = g * 2 + b
                    gbuf, gsem = gbufs[b], gsems[b]
                    for grp in range(BE // 16):
                        sg = srcc_v[i, pl.ds(grp * 16, 16)] - q * NP
                        dg = dstc_v[i, pl.ds(grp * 16, 16)]
                        for h in range(hh):
                            asg = plsc.load_gather(as_v, [sg * hh + h])
                            adg = plsc.load_gather(ad_v, [dg * hh + h])
                            al = asg + adg
                            al = jnp.maximum(al, NEG * al)
                            exv = jnp.exp(al - msp_v[h, :])
                            exb_v[pl.ds(h * BE + grp * 16, 16)] = exv
                            plsc.addupdate_scatter(
                                den_v, [dg * hh + h], exv)

                    sbuf, ssem = sbufs[b], ssems[b]
                    pltpu.make_async_copy(
                        tbl_hbm.at[srcc_v.at[i]], gbuf, gsem).wait()

                    @pl.when(i >= 2)
                    def _():
                        pltpu.make_async_copy(
                            sbuf, acc.at[dstc_v.at[i]], ssem).wait()

                    def e_body(j, ecarry):
                        for u in range(4):
                            e = j * 4 + u
                            svs = [
                                plsc.load_gather(
                                    exb_v,
                                    [jnp.full((16,), h * BE, jnp.int32) + e])
                                for h in range(hh)
                            ]
                            for k in range(nfv):
                                hsel = k // 2 if hh > 1 else 0
                                sbuf[e, pl.ds(k * 16, 16)] = (
                                    gbuf[e, pl.ds(k * 16, 16)] * svs[hsel])
                        return ecarry

                    lax.fori_loop(0, BE // 4, e_body, 0)

                    @pl.when(i + 2 < NB)
                    def _():
                        pltpu.async_copy(
                            tbl_hbm.at[srcc_v.at[i + 2]], gbuf, gsem)
                    pltpu.async_copy(
                        sbuf, acc.at[dstc_v.at[i]], ssem, add=True)
                return carry

            lax.fori_loop(0, NB // 2, outer, 0)
            for b in range(2):
                pltpu.make_async_copy(
                    sbufs[b], acc.at[dstc_v.at[b]], ssems[b]).wait()
            pltpu.sync_copy(den_v, outden_hbm.at[q, s])
            plsc.subcore_barrier()
            pltpu.sync_copy(acc.at[pl.ds(s * RPT, RPT)],
                            out_hbm.at[q, pl.ds(s * RPT, RPT)])

    return edge_kernel


_edge_l1 = _make_edge_kernel(32, 1, 32, 4)
_edge_l2 = _make_edge_kernel(32, 1, 32, 1)


def _lrelu_scalar(m):
    return jnp.where(m > 0, m, NEG * m)


def kernel(x, edge_index, W1, a_src1, a_dst1, b1, W2, a_src2, a_dst2, b2):
    f32 = jnp.float32
    loop = jnp.arange(NNODE, dtype=jnp.int32)
    padi = jnp.full((EP - NNODE - NEDGE,), NNODE, jnp.int32)
    src = jnp.concatenate([edge_index[0].astype(jnp.int32), loop, padi])
    dst = jnp.concatenate([edge_index[1].astype(jnp.int32), loop, padi])
    srcr = src.reshape(16, NB, BE)
    dstg = dst.reshape(16, NB, BE)

    heads = jnp.arange(256, dtype=jnp.int32) // 32
    oh = (heads[:, None] == jnp.arange(8)[None, :]).astype(f32)
    a1 = jnp.concatenate(
        [oh * a_src1.reshape(-1, 1), oh * a_dst1.reshape(-1, 1)], axis=1)
    x_pad = jnp.pad(x, ((0, NP - NNODE), (0, 0)))
    tbl1, asad1 = _dense1(x_pad, W1, a1)
    as1 = asad1[:, 0:8]
    ad1 = asad1[:, 8:16]
    lm1 = _lrelu_scalar(jnp.max(as1, axis=0) + jnp.max(ad1, axis=0))
    msp1 = jnp.broadcast_to(lm1.reshape(8, 1, 1), (8, 1, 16))
    as_t1 = as1.transpose(1, 0)
    ad_t1 = ad1.transpose(1, 0)
    tblf1 = tbl1.reshape(8 * NP, 32)

    acc1, den1 = _edge_l1(srcr, dstg, tblf1, as_t1, ad_t1, msp1)

    a2 = jnp.concatenate(
        [a_src2.reshape(-1, 1), a_dst2.reshape(-1, 1)], axis=1)
    tbl2, asad2 = _dense2(acc1, den1, b1.reshape(1, 256), W2, a2)
    as2 = asad2[:, 0]
    ad2 = asad2[:, 1]
    lm2 = _lrelu_scalar(jnp.max(as2) + jnp.max(ad2)).reshape(1)
    msp2 = jnp.broadcast_to(lm2.reshape(1, 1, 1), (2, 1, 16))
    as_t2 = jnp.broadcast_to(as2.reshape(1, NP), (2, NP))
    ad_t2 = jnp.broadcast_to(ad2.reshape(1, NP), (2, NP))

    acc2, den2 = _edge_l2(srcr, dstg, tbl2.reshape(2 * NP, 32),
                          as_t2, ad_t2, msp2)

    return _finalize(acc2, den2, b2.reshape(1, 64))[:NNODE]
